# Initial kernel scaffold; baseline (speedup 1.0000x reference)
import jax
import jax.numpy as jnp
from jax.experimental import pallas as pl


def kernel(x, pw, w1, b1, gamma1, beta1, w2p, b2p, g2p, be2p):
    raise NotImplementedError("write your pallas kernel here")


# speedup vs baseline: 2.0530x; 2.0530x over previous
import jax
import jax.numpy as jnp
from jax import lax
from jax.experimental import pallas as pl
from jax.experimental.pallas import tpu as pltpu

_OUT = 9
_LANE = 128
_BN_EPS = 1e-5
_SQRT_HALF = 0.7071067811865476
_VMEM_LIMIT = 64 * 1024 * 1024
_POOL_TILE_BYTES = 8 * 1024 * 1024


def _pool_body(q_ref, x_ref, out_ref):
    xb = x_ref[...].astype(jnp.bfloat16)
    s = lax.dot_general(q_ref[...], xb, (((2,), (2,)), ((0,), (0,))),
                        preferred_element_type=jnp.float32)
    m = s.max(axis=-1, keepdims=True)
    p = jnp.exp(s - m)
    denom = p.sum(axis=-1, keepdims=True)
    acc = lax.dot_general(p.astype(jnp.bfloat16), xb, (((2,), (1,)), ((0,), (0,))),
                          preferred_element_type=jnp.float32)
    out_ref[...] = acc / denom


def _head_body(pooled_ref, w1_ref, g1_ref, be1_ref, w2_ref, g2_ref, be2_ref,
               out_ref):
    h = jnp.dot(pooled_ref[...].astype(jnp.bfloat16), w1_ref[...],
                preferred_element_type=jnp.float32)
    mu = jnp.mean(h, axis=0, keepdims=True)
    d = h - mu
    var = jnp.mean(d * d, axis=0, keepdims=True)
    h = d * lax.rsqrt(var + _BN_EPS) * g1_ref[...] + be1_ref[...]
    h = 0.5 * h * (1.0 + lax.erf(h * _SQRT_HALF))

    y = jnp.dot(h.astype(jnp.bfloat16), w2_ref[...],
                preferred_element_type=jnp.float32)
    mu2 = jnp.mean(y, axis=0, keepdims=True)
    d2 = y - mu2
    var2 = jnp.mean(d2 * d2, axis=0, keepdims=True)
    y = d2 * lax.rsqrt(var2 + _BN_EPS) * g2_ref[...] + be2_ref[...]
    out_ref[...] = jax.nn.sigmoid(y)


def _pick_batch_tile(B, S, F):
    row_bytes = S * F * 4
    for tb in (8, 4, 2, 1):
        if B % tb == 0 and tb * row_bytes <= _POOL_TILE_BYTES:
            return tb
    return 1


def kernel(x, pw, w1, b1, gamma1, beta1, w2p, b2p, g2p, be2p):
    B, S, F = x.shape
    tb = _pick_batch_tile(B, S, F)
    wq = jnp.broadcast_to(pw.reshape(1, 1, F), (tb, 1, F))

    pooled = pl.pallas_call(
        _pool_body,
        out_shape=jax.ShapeDtypeStruct((B, 1, F), jnp.float32),
        grid=(B // tb,),
        in_specs=[
            pl.BlockSpec((tb, 1, F), lambda b: (0, 0, 0)),
            pl.BlockSpec((tb, S, F), lambda b: (b, 0, 0)),
        ],
        out_specs=pl.BlockSpec((tb, 1, F), lambda b: (b, 0, 0)),
        compiler_params=pltpu.CompilerParams(
            dimension_semantics=("parallel",),
            vmem_limit_bytes=_VMEM_LIMIT),
    )(wq, x)

    vmem = pl.BlockSpec(memory_space=pltpu.MemorySpace.VMEM)
    out_pad = pl.pallas_call(
        _head_body,
        out_shape=jax.ShapeDtypeStruct((B, _LANE), jnp.float32),
        in_specs=[vmem] * 7,
        out_specs=vmem,
        compiler_params=pltpu.CompilerParams(vmem_limit_bytes=_VMEM_LIMIT),
    )(pooled.reshape(B, F), w1, gamma1, beta1, w2p, g2p, be2p)

    return {'high': out_pad[:, :_OUT]}


# speedup vs baseline: 2.1813x; 1.0625x over previous
import jax
import jax.numpy as jnp
from jax import lax
from jax.experimental import pallas as pl
from jax.experimental.pallas import tpu as pltpu

_OUT = 9
_LANE = 128
_BN_EPS = 1e-5
_SQRT_HALF = 0.7071067811865476
_VMEM_LIMIT = 64 * 1024 * 1024
_POOL_TILE_BYTES = 16 * 1024 * 1024


def _pool_body(q_ref, x_ref, out_ref):
    xb = x_ref[...].astype(jnp.bfloat16)
    s = lax.dot_general(q_ref[...], xb, (((2,), (2,)), ((0,), (0,))),
                        preferred_element_type=jnp.float32)
    m = s.max(axis=-1, keepdims=True)
    p = jnp.exp(s - m)
    denom = p.sum(axis=-1, keepdims=True)
    acc = lax.dot_general(p.astype(jnp.bfloat16), xb, (((2,), (1,)), ((0,), (0,))),
                          preferred_element_type=jnp.float32)
    out_ref[...] = acc / denom


def _head_body(pooled_ref, w1_ref, g1_ref, be1_ref, w2_ref, g2_ref, be2_ref,
               out_ref):
    h = jnp.dot(pooled_ref[...].astype(jnp.bfloat16), w1_ref[...],
                preferred_element_type=jnp.float32)
    mu = jnp.mean(h, axis=0, keepdims=True)
    d = h - mu
    var = jnp.mean(d * d, axis=0, keepdims=True)
    h = d * lax.rsqrt(var + _BN_EPS) * g1_ref[...] + be1_ref[...]
    h = 0.5 * h * (1.0 + lax.erf(h * _SQRT_HALF))

    y = jnp.dot(h.astype(jnp.bfloat16), w2_ref[...],
                preferred_element_type=jnp.float32)
    mu2 = jnp.mean(y, axis=0, keepdims=True)
    d2 = y - mu2
    var2 = jnp.mean(d2 * d2, axis=0, keepdims=True)
    y = d2 * lax.rsqrt(var2 + _BN_EPS) * g2_ref[...] + be2_ref[...]
    out_ref[...] = jax.nn.sigmoid(y)


def _pick_batch_tile(B, S, F):
    row_bytes = S * F * 4
    for tb in (8, 4, 2, 1):
        if B % tb == 0 and tb * row_bytes <= _POOL_TILE_BYTES:
            return tb
    return 1


def kernel(x, pw, w1, b1, gamma1, beta1, w2p, b2p, g2p, be2p):
    B, S, F = x.shape
    tb = _pick_batch_tile(B, S, F)
    wq = jnp.broadcast_to(pw.reshape(1, 1, F), (tb, 1, F))

    pooled = pl.pallas_call(
        _pool_body,
        out_shape=jax.ShapeDtypeStruct((B, 1, F), jnp.float32),
        grid=(B // tb,),
        in_specs=[
            pl.BlockSpec((tb, 1, F), lambda b: (0, 0, 0)),
            pl.BlockSpec((tb, S, F), lambda b: (b, 0, 0)),
        ],
        out_specs=pl.BlockSpec((tb, 1, F), lambda b: (b, 0, 0)),
        compiler_params=pltpu.CompilerParams(
            dimension_semantics=("parallel",),
            vmem_limit_bytes=_VMEM_LIMIT),
    )(wq, x)

    vmem = pl.BlockSpec(memory_space=pltpu.MemorySpace.VMEM)
    out_pad = pl.pallas_call(
        _head_body,
        out_shape=jax.ShapeDtypeStruct((B, _LANE), jnp.float32),
        in_specs=[vmem] * 7,
        out_specs=vmem,
        compiler_params=pltpu.CompilerParams(vmem_limit_bytes=_VMEM_LIMIT),
    )(pooled.reshape(B, F), w1, gamma1, beta1, w2p, g2p, be2p)

    return {'high': out_pad[:, :_OUT]}


# speedup vs baseline: 2.1857x; 1.0020x over previous
import jax
import jax.numpy as jnp
from jax import lax
from jax.experimental import pallas as pl
from jax.experimental.pallas import tpu as pltpu

_OUT = 9
_LANE = 128
_BN_EPS = 1e-5
_SQRT_HALF = 0.7071067811865476
_VMEM_LIMIT = 64 * 1024 * 1024
_POOL_TILE_BYTES = 16 * 1024 * 1024


def _pool_body(q_ref, x_ref, out_ref):
    xb = x_ref[...].astype(jnp.bfloat16)
    s = lax.dot_general(q_ref[...], xb, (((2,), (2,)), ((0,), (0,))),
                        preferred_element_type=jnp.float32)
    m = s.max(axis=-1, keepdims=True)
    p = jnp.exp(s - m)
    denom = p.sum(axis=-1, keepdims=True)
    acc = lax.dot_general(p.astype(jnp.bfloat16), xb, (((2,), (1,)), ((0,), (0,))),
                          preferred_element_type=jnp.float32)
    out_ref[...] = (acc / denom).astype(jnp.bfloat16)


def _head_body(pooled_ref, w1_ref, g1_ref, be1_ref, w2_ref, g2_ref, be2_ref,
               out_ref):
    h = jnp.dot(pooled_ref[...], w1_ref[...],
                preferred_element_type=jnp.float32)
    mu = jnp.mean(h, axis=0, keepdims=True)
    d = h - mu
    var = jnp.mean(d * d, axis=0, keepdims=True)
    h = d * lax.rsqrt(var + _BN_EPS) * g1_ref[...] + be1_ref[...]
    h = 0.5 * h * (1.0 + lax.erf(h * _SQRT_HALF))

    y = jnp.dot(h.astype(jnp.bfloat16), w2_ref[...],
                preferred_element_type=jnp.float32)
    mu2 = jnp.mean(y, axis=0, keepdims=True)
    d2 = y - mu2
    var2 = jnp.mean(d2 * d2, axis=0, keepdims=True)
    y = d2 * lax.rsqrt(var2 + _BN_EPS) * g2_ref[...] + be2_ref[...]
    out_ref[...] = jax.nn.sigmoid(y)


def _pick_batch_tile(B, S, F):
    row_bytes = S * F * 4
    for tb in (8, 4, 2, 1):
        if B % tb == 0 and tb * row_bytes <= _POOL_TILE_BYTES:
            return tb
    return 1


def kernel(x, pw, w1, b1, gamma1, beta1, w2p, b2p, g2p, be2p):
    B, S, F = x.shape
    tb = _pick_batch_tile(B, S, F)
    wq = jnp.broadcast_to(pw.reshape(1, 1, F), (tb, 1, F))

    pooled = pl.pallas_call(
        _pool_body,
        out_shape=jax.ShapeDtypeStruct((B, 1, F), jnp.bfloat16),
        grid=(B // tb,),
        in_specs=[
            pl.BlockSpec((tb, 1, F), lambda b: (0, 0, 0)),
            pl.BlockSpec((tb, S, F), lambda b: (b, 0, 0)),
        ],
        out_specs=pl.BlockSpec((tb, 1, F), lambda b: (b, 0, 0)),
        compiler_params=pltpu.CompilerParams(
            dimension_semantics=("parallel",),
            vmem_limit_bytes=_VMEM_LIMIT),
    )(wq, x)

    vmem = pl.BlockSpec(memory_space=pltpu.MemorySpace.VMEM)
    out_pad = pl.pallas_call(
        _head_body,
        out_shape=jax.ShapeDtypeStruct((B, _LANE), jnp.float32),
        in_specs=[vmem] * 7,
        out_specs=vmem,
        compiler_params=pltpu.CompilerParams(vmem_limit_bytes=_VMEM_LIMIT),
    )(pooled.reshape(B, F), w1, gamma1, beta1, w2p, g2p, be2p)

    return {'high': out_pad[:, :_OUT]}
